# baseline (speedup 1.0000x reference)
import functools

import jax
import jax.numpy as jnp
from jax import lax
from jax.experimental import pallas as pl
from jax.experimental.pallas import tpu as pltpu
from jax.experimental.pallas import tpu_sc as plsc

NC = 2
NS = 16
NW = NC * NS
LN = 16

_MESH = plsc.VectorSubcoreMesh(
    core_axis_name="c", subcore_axis_name="s", num_cores=NC, num_subcores=NS
)
_PARAMS = pltpu.CompilerParams(needs_layout_passes=False)


def _wid():
    return lax.axis_index("s") * NC + lax.axis_index("c")


def _make_edge_hist(n_nodes: int, n_edges: int, n_graphs: int):
    blk = 128
    nblk = n_edges // blk
    assert n_edges % blk == 0
    nb = nblk // NW
    rem = nblk % NW
    nchunks = 10
    cblk = nb // nchunks
    cw = cblk * blk
    unroll = 8
    assert nb % nchunks == 0 and (cw // LN) % unroll == 0
    hstride = n_graphs + 1
    hwords = (LN * hstride + 127) // 128 * 128

    @functools.partial(
        pl.kernel,
        out_type=jax.ShapeDtypeStruct((NW, n_graphs), jnp.int32),
        mesh=_MESH,
        compiler_params=_PARAMS,
        scratch_types=[
            pltpu.VMEM((n_nodes,), jnp.int32),
            pltpu.VMEM((2, cw), jnp.int32),
            pltpu.VMEM((2, cw), jnp.int32),
            pltpu.VMEM((hwords,), jnp.int32),
            pltpu.VMEM((n_graphs,), jnp.int32),
            pltpu.SemaphoreType.DMA,
            pltpu.SemaphoreType.DMA,
        ],
    )
    def edge_hist(batch_hbm, ei_hbm, out_hbm, batch_v, ebuf_a, ebuf_b,
                  hist_v, row_v, sem_a, sem_b):
        wid = _wid()
        zeros = jnp.zeros((LN,), jnp.int32)
        ones = jnp.ones((LN,), jnp.int32)
        lane = lax.iota(jnp.int32, LN)
        lane_g = lane * hstride

        @plsc.parallel_loop(0, hwords // LN, 1, unroll=8)
        def _(i):
            hist_v[pl.ds(i * LN, LN)] = zeros

        col0 = (wid * nb + jnp.minimum(wid, rem)) * blk

        def start(buf, sem, c):
            pltpu.async_copy(ei_hbm.at[:, pl.ds(col0 + c * cw, cw)], buf, sem)

        def wait(buf, sem):
            pltpu.make_async_copy(ei_hbm.at[:, pl.ds(col0, cw)], buf, sem).wait()

        def hist16(idx):
            vals = plsc.load_gather(batch_v, [idx])
            plsc.addupdate_scatter(hist_v, [lane_g + vals], ones)

        def process(buf):
            @plsc.parallel_loop(0, cw // LN, 1, unroll=unroll)
            def _(i):
                hist16(buf[1, pl.ds(i * LN, LN)])

        start(ebuf_a, sem_a, 0)
        pltpu.sync_copy(batch_hbm, batch_v)
        start(ebuf_b, sem_b, 1)

        def pair(p, _):
            wait(ebuf_a, sem_a)
            process(ebuf_a)
            start(ebuf_a, sem_a, 2 * p + 2)
            wait(ebuf_b, sem_b)
            process(ebuf_b)
            start(ebuf_b, sem_b, 2 * p + 3)
            return 0

        lax.fori_loop(0, nchunks // 2 - 1, pair, 0)
        wait(ebuf_a, sem_a)
        process(ebuf_a)
        wait(ebuf_b, sem_b)
        process(ebuf_b)

        @pl.when(wid < rem)
        def _():
            pltpu.sync_copy(
                ei_hbm.at[:, pl.ds(col0 + nb * blk, blk)],
                ebuf_a.at[:, pl.ds(0, blk)],
            )
            for u in range(blk // LN):
                hist16(ebuf_a[1, pl.ds(u * LN, LN)])

        def red_body(g, _):
            acc = zeros
            for l in range(LN):
                acc = acc + hist_v[pl.ds(l * hstride + g * LN, LN)]
            row_v[pl.ds(g * LN, LN)] = acc
            return 0

        lax.fori_loop(0, n_graphs // LN, red_body, 0)
        pltpu.sync_copy(row_v, out_hbm.at[wid])

    return edge_hist


def _make_dense_tc(n_nodes: int, n_attr: int):
    bn = 2048
    grid = (n_nodes + bn - 1) // bn

    def body(attrs_ref, energy_ref, level_ref, st_ref, ht_ref, out_ref):
        a = attrs_ref[...]
        sa = jnp.dot(a, st_ref[...], preferred_element_type=jnp.float32,
                     precision=lax.Precision.HIGHEST)
        ha = jnp.dot(a, ht_ref[...], preferred_element_type=jnp.float32,
                     precision=lax.Precision.HIGHEST)
        lvl = level_ref[...][:, None]
        s = jnp.where(lvl == 0, sa[:, 0:1], sa[:, 1:2])
        h = jnp.where(lvl == 0, ha[:, 0:1], ha[:, 1:2])
        t = energy_ref[...][:, None] * s + h
        out_ref[...] = t[:, 0]

    return pl.pallas_call(
        body,
        grid=(grid,),
        in_specs=[
            pl.BlockSpec((bn, n_attr), lambda i: (i, 0)),
            pl.BlockSpec((bn,), lambda i: (i,)),
            pl.BlockSpec((bn,), lambda i: (i,)),
            pl.BlockSpec((n_attr, 2), lambda i: (0, 0)),
            pl.BlockSpec((n_attr, 2), lambda i: (0, 0)),
        ],
        out_specs=pl.BlockSpec((bn,), lambda i: (i,)),
        out_shape=jax.ShapeDtypeStruct((n_nodes,), jnp.float32),
    )


def _make_mask_apply(n_nodes: int, n_graphs: int, ptr_pad: int):
    npt = (n_nodes // NW) // LN * LN
    tail = n_nodes - NW * npt
    assert npt % 8 == 0 and tail % LN == 0
    nbuf = npt + tail

    @functools.partial(
        pl.kernel,
        out_type=jax.ShapeDtypeStruct((n_nodes,), jnp.float32),
        mesh=_MESH,
        compiler_params=_PARAMS,
        scratch_types=[
            pltpu.VMEM((NW, n_graphs), jnp.int32),
            pltpu.VMEM((n_graphs,), jnp.int32),
            pltpu.VMEM((ptr_pad,), jnp.int32),
            pltpu.VMEM((nbuf,), jnp.int32),
            pltpu.VMEM((nbuf,), jnp.float32),
            pltpu.VMEM((nbuf,), jnp.float32),
            pltpu.SemaphoreType.DMA,
        ],
    )
    def mask_apply(
        part_hbm, ptr_hbm, batch_hbm, t_hbm, out_hbm,
        part_v, mask_v, ptr_v, batch_v, t_v, out_v, sem,
    ):
        wid = _wid()
        zeros = jnp.zeros((LN,), jnp.int32)
        fzeros = jnp.zeros((LN,), jnp.float32)

        pltpu.sync_copy(part_hbm, part_v)
        pltpu.sync_copy(ptr_hbm, ptr_v)

        def mask_body(g, _):
            ne = zeros
            for r in range(NW):
                ne = ne + part_v[r, pl.ds(g * LN, LN)]
            nn = ptr_v[pl.ds(g * LN + 1, LN)] - ptr_v[pl.ds(g * LN, LN)]
            iso = ((nn == 1) & (ne == 0)).astype(jnp.int32)
            mask_v[pl.ds(g * LN, LN)] = iso
            return 0

        lax.fori_loop(0, n_graphs // LN, mask_body, 0)

        nbase = wid * npt
        pltpu.sync_copy(batch_hbm.at[pl.ds(nbase, npt)], batch_v.at[pl.ds(0, npt)])
        pltpu.sync_copy(t_hbm.at[pl.ds(nbase, npt)], t_v.at[pl.ds(0, npt)])

        tbase = NW * npt

        @pl.when(wid == NW - 1)
        def _():
            pltpu.sync_copy(
                batch_hbm.at[pl.ds(tbase, tail)], batch_v.at[pl.ds(npt, tail)]
            )
            pltpu.sync_copy(t_hbm.at[pl.ds(tbase, tail)], t_v.at[pl.ds(npt, tail)])

        def node_body(j):
            sl = pl.ds(j * LN, LN)
            iso = plsc.load_gather(mask_v, [batch_v[sl]])
            out_v[sl] = jnp.where(iso == 1, fzeros, t_v[sl])

        @plsc.parallel_loop(0, npt // LN, 1, unroll=5)
        def _(j):
            node_body(j)

        @pl.when(wid == NW - 1)
        def _():
            @plsc.parallel_loop(npt // LN, nbuf // LN, 1, unroll=2)
            def _(j):
                node_body(j)

        pltpu.sync_copy(out_v.at[pl.ds(0, npt)], out_hbm.at[pl.ds(nbase, npt)])

        @pl.when(wid == NW - 1)
        def _():
            pltpu.sync_copy(
                out_v.at[pl.ds(npt, tail)], out_hbm.at[pl.ds(tbase, tail)]
            )

    return mask_apply


def kernel(node_energy, node_attrs, ptr, edge_index, batch, node_level, scale, shift):
    n_nodes = node_energy.shape[0]
    n_edges = edge_index.shape[1]
    n_graphs = ptr.shape[0] - 1
    n_attr = node_attrs.shape[1]

    ptr_pad = (ptr.shape[0] + 15) // 16 * 16
    ptr_p = jnp.pad(ptr, (0, ptr_pad - ptr.shape[0]))

    partials = _make_edge_hist(n_nodes, n_edges, n_graphs)(batch, edge_index)
    t = _make_dense_tc(n_nodes, n_attr)(
        node_attrs, node_energy, node_level, scale.T, shift.T
    )
    out = _make_mask_apply(n_nodes, n_graphs, ptr_pad)(
        partials, ptr_p, batch, t
    )
    return out


# speedup vs baseline: 1.2046x; 1.2046x over previous
import functools

import jax
import jax.numpy as jnp
from jax import lax
from jax.experimental import pallas as pl
from jax.experimental.pallas import tpu as pltpu
from jax.experimental.pallas import tpu_sc as plsc

NC = 2
NS = 16
NW = NC * NS
LN = 16

_MESH = plsc.VectorSubcoreMesh(
    core_axis_name="c", subcore_axis_name="s", num_cores=NC, num_subcores=NS
)
_PARAMS = pltpu.CompilerParams(needs_layout_passes=False)


def _wid():
    return lax.axis_index("s") * NC + lax.axis_index("c")


def _make_edge_hist(n_nodes: int, n_edges: int, n_graphs: int):
    blk = 128
    nblk = n_edges // blk
    assert n_edges % blk == 0
    nb = nblk // NW
    rem = nblk % NW
    nchunks = 10
    cblk = nb // nchunks
    cw = cblk * blk
    unroll = 8
    assert nb % nchunks == 0 and (cw // LN) % unroll == 0
    hstride = n_graphs + 1
    hwords = (LN * hstride + 127) // 128 * 128

    @functools.partial(
        pl.kernel,
        out_type=jax.ShapeDtypeStruct((NW, n_graphs), jnp.int32),
        mesh=_MESH,
        compiler_params=_PARAMS,
        scratch_types=[
            pltpu.VMEM((n_nodes,), jnp.int32),
            pltpu.VMEM((2, cw), jnp.int32),
            pltpu.VMEM((2, cw), jnp.int32),
            pltpu.VMEM((hwords,), jnp.int32),
            pltpu.VMEM((n_graphs,), jnp.int32),
            pltpu.SemaphoreType.DMA,
            pltpu.SemaphoreType.DMA,
        ],
    )
    def edge_hist(batch_hbm, ei_hbm, out_hbm, batch_v, ebuf_a, ebuf_b,
                  hist_v, row_v, sem_a, sem_b):
        wid = _wid()
        zeros = jnp.zeros((LN,), jnp.int32)
        ones = jnp.ones((LN,), jnp.int32)
        lane = lax.iota(jnp.int32, LN)
        lane_g = lane * hstride

        @plsc.parallel_loop(0, hwords // LN, 1, unroll=8)
        def _(i):
            hist_v[pl.ds(i * LN, LN)] = zeros

        col0 = (wid * nb + jnp.minimum(wid, rem)) * blk

        def start(buf, sem, c):
            pltpu.async_copy(ei_hbm.at[:, pl.ds(col0 + c * cw, cw)], buf, sem)

        def wait(buf, sem):
            pltpu.make_async_copy(ei_hbm.at[:, pl.ds(col0, cw)], buf, sem).wait()

        def hist16(idx):
            vals = plsc.load_gather(batch_v, [idx])
            plsc.addupdate_scatter(hist_v, [lane_g + vals], ones)

        def process(buf):
            @plsc.parallel_loop(0, cw // LN, 1, unroll=unroll)
            def _(i):
                hist16(buf[1, pl.ds(i * LN, LN)])

        start(ebuf_a, sem_a, 0)
        pltpu.sync_copy(batch_hbm, batch_v)
        start(ebuf_b, sem_b, 1)

        def pair(p, _):
            wait(ebuf_a, sem_a)
            process(ebuf_a)
            start(ebuf_a, sem_a, 2 * p + 2)
            wait(ebuf_b, sem_b)
            process(ebuf_b)
            start(ebuf_b, sem_b, 2 * p + 3)
            return 0

        lax.fori_loop(0, nchunks // 2 - 1, pair, 0)
        wait(ebuf_a, sem_a)
        process(ebuf_a)
        wait(ebuf_b, sem_b)
        process(ebuf_b)

        @pl.when(wid < rem)
        def _():
            pltpu.sync_copy(
                ei_hbm.at[:, pl.ds(col0 + nb * blk, blk)],
                ebuf_a.at[:, pl.ds(0, blk)],
            )
            for u in range(blk // LN):
                hist16(ebuf_a[1, pl.ds(u * LN, LN)])

        def red_body(g, _):
            acc = zeros
            for l in range(LN):
                acc = acc + hist_v[pl.ds(l * hstride + g * LN, LN)]
            row_v[pl.ds(g * LN, LN)] = acc
            return 0

        lax.fori_loop(0, n_graphs // LN, red_body, 0)
        pltpu.sync_copy(row_v, out_hbm.at[wid])

    return edge_hist


def _make_dense_tc(n_nodes: int, n_attr: int, n_lvl: int):
    bn = 2000
    assert n_nodes % bn == 0
    grid = n_nodes // bn
    dn = (((1,), (1,)), ((), ()))

    def body(attrs_ref, energy_ref, level_ref, s_ref, h_ref, out_ref):
        a = attrs_ref[...]
        sa = lax.dot_general(s_ref[...], a, dn,
                             preferred_element_type=jnp.float32,
                             precision=lax.Precision.HIGHEST)
        ha = lax.dot_general(h_ref[...], a, dn,
                             preferred_element_type=jnp.float32,
                             precision=lax.Precision.HIGHEST)
        lvl = level_ref[0]
        s = jnp.where(lvl == 0, sa[0:1, :], sa[1:2, :])
        h = jnp.where(lvl == 0, ha[0:1, :], ha[1:2, :])
        out_ref[0] = energy_ref[0] * s + h

    return pl.pallas_call(
        body,
        grid=(grid,),
        in_specs=[
            pl.BlockSpec((bn, n_attr), lambda i: (i, 0)),
            pl.BlockSpec((1, 1, bn), lambda i: (i, 0, 0)),
            pl.BlockSpec((1, 1, bn), lambda i: (i, 0, 0)),
            pl.BlockSpec((n_lvl, n_attr), lambda i: (0, 0)),
            pl.BlockSpec((n_lvl, n_attr), lambda i: (0, 0)),
        ],
        out_specs=pl.BlockSpec((1, 1, bn), lambda i: (i, 0, 0)),
        out_shape=jax.ShapeDtypeStruct((grid, 1, bn), jnp.float32),
    )


def _make_mask_apply(n_nodes: int, n_graphs: int, ptr_pad: int):
    npt = (n_nodes // NW) // LN * LN
    tail = n_nodes - NW * npt
    assert npt % 8 == 0 and tail % LN == 0
    nbuf = npt + tail

    @functools.partial(
        pl.kernel,
        out_type=jax.ShapeDtypeStruct((n_nodes,), jnp.float32),
        mesh=_MESH,
        compiler_params=_PARAMS,
        scratch_types=[
            pltpu.VMEM((NW, n_graphs), jnp.int32),
            pltpu.VMEM((n_graphs,), jnp.int32),
            pltpu.VMEM((ptr_pad,), jnp.int32),
            pltpu.VMEM((nbuf,), jnp.int32),
            pltpu.VMEM((nbuf,), jnp.float32),
            pltpu.VMEM((nbuf,), jnp.float32),
            pltpu.SemaphoreType.DMA,
        ],
    )
    def mask_apply(
        part_hbm, ptr_hbm, batch_hbm, t_hbm, out_hbm,
        part_v, mask_v, ptr_v, batch_v, t_v, out_v, sem,
    ):
        wid = _wid()
        zeros = jnp.zeros((LN,), jnp.int32)
        fzeros = jnp.zeros((LN,), jnp.float32)

        pltpu.sync_copy(part_hbm, part_v)
        pltpu.sync_copy(ptr_hbm, ptr_v)

        def mask_body(g, _):
            ne = zeros
            for r in range(NW):
                ne = ne + part_v[r, pl.ds(g * LN, LN)]
            nn = ptr_v[pl.ds(g * LN + 1, LN)] - ptr_v[pl.ds(g * LN, LN)]
            iso = ((nn == 1) & (ne == 0)).astype(jnp.int32)
            mask_v[pl.ds(g * LN, LN)] = iso
            return 0

        lax.fori_loop(0, n_graphs // LN, mask_body, 0)

        nbase = wid * npt
        pltpu.sync_copy(batch_hbm.at[pl.ds(nbase, npt)], batch_v.at[pl.ds(0, npt)])
        pltpu.sync_copy(t_hbm.at[pl.ds(nbase, npt)], t_v.at[pl.ds(0, npt)])

        tbase = NW * npt

        @pl.when(wid == NW - 1)
        def _():
            pltpu.sync_copy(
                batch_hbm.at[pl.ds(tbase, tail)], batch_v.at[pl.ds(npt, tail)]
            )
            pltpu.sync_copy(t_hbm.at[pl.ds(tbase, tail)], t_v.at[pl.ds(npt, tail)])

        def node_body(j):
            sl = pl.ds(j * LN, LN)
            iso = plsc.load_gather(mask_v, [batch_v[sl]])
            out_v[sl] = jnp.where(iso == 1, fzeros, t_v[sl])

        @plsc.parallel_loop(0, npt // LN, 1, unroll=5)
        def _(j):
            node_body(j)

        @pl.when(wid == NW - 1)
        def _():
            @plsc.parallel_loop(npt // LN, nbuf // LN, 1, unroll=2)
            def _(j):
                node_body(j)

        pltpu.sync_copy(out_v.at[pl.ds(0, npt)], out_hbm.at[pl.ds(nbase, npt)])

        @pl.when(wid == NW - 1)
        def _():
            pltpu.sync_copy(
                out_v.at[pl.ds(npt, tail)], out_hbm.at[pl.ds(tbase, tail)]
            )

    return mask_apply


def kernel(node_energy, node_attrs, ptr, edge_index, batch, node_level, scale, shift):
    n_nodes = node_energy.shape[0]
    n_edges = edge_index.shape[1]
    n_graphs = ptr.shape[0] - 1
    n_attr = node_attrs.shape[1]

    ptr_pad = (ptr.shape[0] + 15) // 16 * 16
    ptr_p = jnp.pad(ptr, (0, ptr_pad - ptr.shape[0]))

    partials = _make_edge_hist(n_nodes, n_edges, n_graphs)(batch, edge_index)
    bn = 2000
    t = _make_dense_tc(n_nodes, n_attr, scale.shape[0])(
        node_attrs,
        node_energy.reshape(n_nodes // bn, 1, bn),
        node_level.reshape(n_nodes // bn, 1, bn),
        scale,
        shift,
    ).reshape(n_nodes)
    out = _make_mask_apply(n_nodes, n_graphs, ptr_pad)(
        partials, ptr_p, batch, t
    )
    return out


# speedup vs baseline: 2.8852x; 2.3952x over previous
import functools

import jax
import jax.numpy as jnp
from jax import lax
from jax.experimental import pallas as pl
from jax.experimental.pallas import tpu as pltpu
from jax.experimental.pallas import tpu_sc as plsc

NC = 2
NS = 16
NW = NC * NS
LN = 16

_MESH = plsc.VectorSubcoreMesh(
    core_axis_name="c", subcore_axis_name="s", num_cores=NC, num_subcores=NS
)
_PARAMS = pltpu.CompilerParams(needs_layout_passes=False)


def _wid():
    return lax.axis_index("s") * NC + lax.axis_index("c")


def _make_edge_hist(n_nodes: int, n_edges: int, n_graphs: int):
    blk = 128
    nblk = n_edges // blk
    assert n_edges % blk == 0
    nb = nblk // NW
    rem = nblk % NW
    nchunks = 10
    cblk = nb // nchunks
    cw = cblk * blk
    unroll = 8
    assert nb % nchunks == 0 and (cw // LN) % unroll == 0
    hstride = n_graphs + 1
    hwords = (LN * hstride + 127) // 128 * 128

    @functools.partial(
        pl.kernel,
        out_type=jax.ShapeDtypeStruct((NW, n_graphs), jnp.int32),
        mesh=_MESH,
        compiler_params=_PARAMS,
        scratch_types=[
            pltpu.VMEM((n_nodes,), jnp.int32),
            pltpu.VMEM((2, cw), jnp.int32),
            pltpu.VMEM((2, cw), jnp.int32),
            pltpu.VMEM((hwords,), jnp.int32),
            pltpu.VMEM((n_graphs,), jnp.int32),
            pltpu.SemaphoreType.DMA,
            pltpu.SemaphoreType.DMA,
        ],
    )
    def edge_hist(batch_hbm, ei_hbm, out_hbm, batch_v, ebuf_a, ebuf_b,
                  hist_v, row_v, sem_a, sem_b):
        wid = _wid()
        zeros = jnp.zeros((LN,), jnp.int32)
        ones = jnp.ones((LN,), jnp.int32)
        lane = lax.iota(jnp.int32, LN)
        lane_g = lane * hstride

        @plsc.parallel_loop(0, hwords // LN, 1, unroll=8)
        def _(i):
            hist_v[pl.ds(i * LN, LN)] = zeros

        col0 = (wid * nb + jnp.minimum(wid, rem)) * blk

        def start(buf, sem, c):
            pltpu.async_copy(ei_hbm.at[:, pl.ds(col0 + c * cw, cw)], buf, sem)

        def wait(buf, sem):
            pltpu.make_async_copy(ei_hbm.at[:, pl.ds(col0, cw)], buf, sem).wait()

        def hist16(idx):
            vals = plsc.load_gather(batch_v, [idx])
            plsc.addupdate_scatter(hist_v, [lane_g + vals], ones)

        def process(buf):
            @plsc.parallel_loop(0, cw // LN, 1, unroll=unroll)
            def _(i):
                hist16(buf[1, pl.ds(i * LN, LN)])

        start(ebuf_a, sem_a, 0)
        pltpu.sync_copy(batch_hbm, batch_v)
        start(ebuf_b, sem_b, 1)

        def pair(p, _):
            wait(ebuf_a, sem_a)
            process(ebuf_a)
            start(ebuf_a, sem_a, 2 * p + 2)
            wait(ebuf_b, sem_b)
            process(ebuf_b)
            start(ebuf_b, sem_b, 2 * p + 3)
            return 0

        lax.fori_loop(0, nchunks // 2 - 1, pair, 0)
        wait(ebuf_a, sem_a)
        process(ebuf_a)
        wait(ebuf_b, sem_b)
        process(ebuf_b)

        @pl.when(wid < rem)
        def _():
            pltpu.sync_copy(
                ei_hbm.at[:, pl.ds(col0 + nb * blk, blk)],
                ebuf_a.at[:, pl.ds(0, blk)],
            )
            for u in range(blk // LN):
                hist16(ebuf_a[1, pl.ds(u * LN, LN)])

        def red_body(g, _):
            acc = zeros
            for l in range(LN):
                acc = acc + hist_v[pl.ds(l * hstride + g * LN, LN)]
            row_v[pl.ds(g * LN, LN)] = acc
            return 0

        lax.fori_loop(0, n_graphs // LN, red_body, 0)
        pltpu.sync_copy(row_v, out_hbm.at[wid])

    return edge_hist


def _make_node_phase(n_nodes: int, n_graphs: int, ptr_pad: int, n_attr: int):
    npt = (n_nodes // NW) // LN * LN
    tail = n_nodes - NW * npt
    assert npt % 8 == 0 and tail % LN == 0
    nbuf = npt + tail

    @functools.partial(
        pl.kernel,
        out_type=jax.ShapeDtypeStruct((n_nodes,), jnp.float32),
        mesh=_MESH,
        compiler_params=_PARAMS,
        scratch_types=[
            pltpu.VMEM((NW, n_graphs), jnp.int32),
            pltpu.VMEM((n_graphs,), jnp.int32),
            pltpu.VMEM((ptr_pad,), jnp.int32),
            pltpu.VMEM((64,), jnp.float32),
            pltpu.VMEM((nbuf,), jnp.int32),
            pltpu.VMEM((nbuf,), jnp.int32),
            pltpu.VMEM((nbuf,), jnp.float32),
            pltpu.VMEM((n_attr * nbuf,), jnp.float32),
            pltpu.VMEM((nbuf,), jnp.float32),
            pltpu.SemaphoreType.DMA,
        ],
    )
    def node_phase(
        part_hbm, ptr_hbm, coef_hbm, batch_hbm, level_hbm, energy_hbm,
        attrs_hbm, out_hbm, part_v, mask_v, ptr_v, coef_v, batch_v, level_v,
        energy_v, attrs_v, out_v, sem,
    ):
        wid = _wid()
        zeros = jnp.zeros((LN,), jnp.int32)
        fzeros = jnp.zeros((LN,), jnp.float32)

        pltpu.sync_copy(part_hbm, part_v)
        pltpu.sync_copy(ptr_hbm, ptr_v)
        pltpu.sync_copy(coef_hbm, coef_v)

        def mask_body(g, _):
            ne = zeros
            for r in range(NW):
                ne = ne + part_v[r, pl.ds(g * LN, LN)]
            nn = ptr_v[pl.ds(g * LN + 1, LN)] - ptr_v[pl.ds(g * LN, LN)]
            iso = ((nn == 1) & (ne == 0)).astype(jnp.int32)
            mask_v[pl.ds(g * LN, LN)] = iso
            return 0

        lax.fori_loop(0, n_graphs // LN, mask_body, 0)

        nbase = wid * npt
        pltpu.sync_copy(batch_hbm.at[pl.ds(nbase, npt)], batch_v.at[pl.ds(0, npt)])
        pltpu.sync_copy(level_hbm.at[pl.ds(nbase, npt)], level_v.at[pl.ds(0, npt)])
        pltpu.sync_copy(energy_hbm.at[pl.ds(nbase, npt)], energy_v.at[pl.ds(0, npt)])
        for z in range(n_attr):
            pltpu.sync_copy(
                attrs_hbm.at[pl.ds(z * n_nodes + nbase, npt)],
                attrs_v.at[pl.ds(z * nbuf, npt)],
            )

        tbase = NW * npt

        @pl.when(wid == NW - 1)
        def _():
            pltpu.sync_copy(
                batch_hbm.at[pl.ds(tbase, tail)], batch_v.at[pl.ds(npt, tail)]
            )
            pltpu.sync_copy(
                level_hbm.at[pl.ds(tbase, tail)], level_v.at[pl.ds(npt, tail)]
            )
            pltpu.sync_copy(
                energy_hbm.at[pl.ds(tbase, tail)], energy_v.at[pl.ds(npt, tail)]
            )
            for z in range(n_attr):
                pltpu.sync_copy(
                    attrs_hbm.at[pl.ds(z * n_nodes + tbase, tail)],
                    attrs_v.at[pl.ds(z * nbuf + npt, tail)],
                )

        s0 = [plsc.load_gather(coef_v, [jnp.full((LN,), z, jnp.int32)])
              for z in range(n_attr)]
        s1 = [plsc.load_gather(coef_v, [jnp.full((LN,), 16 + z, jnp.int32)])
              for z in range(n_attr)]
        h0 = [plsc.load_gather(coef_v, [jnp.full((LN,), 32 + z, jnp.int32)])
              for z in range(n_attr)]
        h1 = [plsc.load_gather(coef_v, [jnp.full((LN,), 48 + z, jnp.int32)])
              for z in range(n_attr)]

        def node_body(j):
            sl = pl.ds(j * LN, LN)
            iso = plsc.load_gather(mask_v, [batch_v[sl]])
            lvlf = level_v[sl].astype(jnp.float32)
            d0 = fzeros
            d1 = fzeros
            e0 = fzeros
            e1 = fzeros
            for z in range(n_attr):
                a = attrs_v[pl.ds(z * nbuf + j * LN, LN)]
                d0 = d0 + a * s0[z]
                d1 = d1 + a * s1[z]
                e0 = e0 + a * h0[z]
                e1 = e1 + a * h1[z]
            ns = d0 + lvlf * (d1 - d0)
            nh = e0 + lvlf * (e1 - e0)
            res = energy_v[sl] * ns + nh
            out_v[sl] = jnp.where(iso == 1, fzeros, res)

        @plsc.parallel_loop(0, npt // LN, 1, unroll=3)
        def _(j):
            node_body(j)

        @pl.when(wid == NW - 1)
        def _():
            @plsc.parallel_loop(npt // LN, nbuf // LN, 1, unroll=2)
            def _(j):
                node_body(j)

        pltpu.sync_copy(out_v.at[pl.ds(0, npt)], out_hbm.at[pl.ds(nbase, npt)])

        @pl.when(wid == NW - 1)
        def _():
            pltpu.sync_copy(
                out_v.at[pl.ds(npt, tail)], out_hbm.at[pl.ds(tbase, tail)]
            )

    return node_phase


def kernel(node_energy, node_attrs, ptr, edge_index, batch, node_level, scale, shift):
    n_nodes = node_energy.shape[0]
    n_edges = edge_index.shape[1]
    n_graphs = ptr.shape[0] - 1
    n_attr = node_attrs.shape[1]

    ptr_pad = (ptr.shape[0] + 15) // 16 * 16
    ptr_p = jnp.pad(ptr, (0, ptr_pad - ptr.shape[0]))
    sc_p = jnp.pad(scale, ((0, 0), (0, 16 - n_attr))).reshape(-1)
    sh_p = jnp.pad(shift, ((0, 0), (0, 16 - n_attr))).reshape(-1)
    coef = jnp.concatenate([sc_p, sh_p])
    attrs_zmajor = node_attrs.T.reshape(-1)

    partials = _make_edge_hist(n_nodes, n_edges, n_graphs)(batch, edge_index)
    out = _make_node_phase(n_nodes, n_graphs, ptr_pad, n_attr)(
        partials, ptr_p, coef, batch, node_level, node_energy, attrs_zmajor
    )
    return out
